# Initial kernel scaffold; baseline (speedup 1.0000x reference)
import jax
import jax.numpy as jnp
from jax.experimental import pallas as pl


def kernel(x, edge_index, W1, b1, W2, b2):
    raise NotImplementedError("write your pallas kernel here")


# speedup vs baseline: 16.1593x; 16.1593x over previous
import functools

import jax
import jax.numpy as jnp
from jax import lax
from jax.experimental import pallas as pl
from jax.experimental.pallas import tpu as pltpu
from jax.experimental.pallas import tpu_sc as plsc

N = 10000
NPAD = 10240
E = 320000
EB = 128
NTILES = 32
EPAD = 323584
E_PER_SC = EPAD // 2
E_PER_TILE = E_PER_SC // 16
BLOCKS = E_PER_TILE // EB
ROWS_PER_TILE = NPAD // 16
OUT_CHUNKS = ROWS_PER_TILE // 128

_mesh = plsc.VectorSubcoreMesh(core_axis_name="c", subcore_axis_name="s")


def _sc_deg(dst_pad, ones128, zeros640):
    @functools.partial(
        pl.kernel,
        out_type=jax.ShapeDtypeStruct((2 * NPAD,), jnp.float32),
        mesh=_mesh,
        scratch_types=[
            pltpu.VMEM((EB,), jnp.int32),
            pltpu.VMEM((EB,), jnp.float32),
            pltpu.VMEM((ROWS_PER_TILE,), jnp.float32),
            pltpu.VMEM_SHARED((NPAD,), jnp.float32),
        ],
    )
    def k(dst_hbm, ones_hbm, zeros_hbm, out_hbm, di_v, ones_v, buf_v, acc_sh):
        c = lax.axis_index("c")
        s = lax.axis_index("s")
        row0 = pl.multiple_of(s * ROWS_PER_TILE, 128)
        pltpu.sync_copy(ones_hbm, ones_v)
        pltpu.sync_copy(zeros_hbm, buf_v)
        pltpu.sync_copy(buf_v, acc_sh.at[pl.ds(row0, ROWS_PER_TILE)])
        plsc.subcore_barrier()
        base = c * E_PER_SC + s * E_PER_TILE

        def body(i, carry):
            off = pl.multiple_of(base + i * EB, EB)
            pltpu.sync_copy(dst_hbm.at[pl.ds(off, EB)], di_v)
            pltpu.sync_copy(ones_v, acc_sh.at[di_v], add=True)
            return carry

        lax.fori_loop(0, BLOCKS, body, 0)
        plsc.subcore_barrier()
        pltpu.sync_copy(acc_sh.at[pl.ds(row0, ROWS_PER_TILE)], buf_v)
        pltpu.sync_copy(buf_v, out_hbm.at[pl.ds(c * NPAD + row0, ROWS_PER_TILE)])

    return k(dst_pad, ones128, zeros640)


def _sc_agg(src_pad, dst_pad, xws, zrows, d_feat):
    @functools.partial(
        pl.kernel,
        out_type=jax.ShapeDtypeStruct((2 * NPAD, d_feat), jnp.float32),
        mesh=_mesh,
        scratch_types=[
            pltpu.VMEM((EB,), jnp.int32),
            pltpu.VMEM((EB,), jnp.int32),
            pltpu.VMEM((EB, d_feat), jnp.float32),
            pltpu.VMEM((EB, d_feat), jnp.float32),
            pltpu.VMEM_SHARED((NPAD, d_feat), jnp.float32),
            pltpu.SemaphoreType.DMA,
        ],
    )
    def k(src_hbm, dst_hbm, xws_hbm, z_hbm, out_hbm,
          si_v, di_v, rows_v, z_v, acc_sh, sem):
        c = lax.axis_index("c")
        s = lax.axis_index("s")
        row0 = pl.multiple_of(s * ROWS_PER_TILE, 128)
        pltpu.sync_copy(z_hbm, z_v)

        def zinit(kk, carry):
            ro = pl.multiple_of(row0 + kk * 128, 128)
            pltpu.sync_copy(z_v, acc_sh.at[pl.ds(ro, 128)])
            return carry

        lax.fori_loop(0, OUT_CHUNKS, zinit, 0)
        plsc.subcore_barrier()
        base = c * E_PER_SC + s * E_PER_TILE

        def body(i, carry):
            off = pl.multiple_of(base + i * EB, EB)
            pltpu.sync_copy(src_hbm.at[pl.ds(off, EB)], si_v)
            pltpu.sync_copy(dst_hbm.at[pl.ds(off, EB)], di_v)
            pltpu.async_copy(xws_hbm.at[si_v], rows_v, sem).wait()
            pltpu.sync_copy(rows_v, acc_sh.at[di_v], add=True)
            return carry

        lax.fori_loop(0, BLOCKS, body, 0)
        plsc.subcore_barrier()

        def cout(kk, carry):
            ro = pl.multiple_of(row0 + kk * 128, 128)
            pltpu.sync_copy(acc_sh.at[pl.ds(ro, 128)], rows_v)
            pltpu.sync_copy(rows_v, out_hbm.at[pl.ds(c * NPAD + ro, 128)])
            return carry

        lax.fori_loop(0, OUT_CHUNKS, cout, 0)

    return k(src_pad, dst_pad, xws, zrows)


_BR = 1024
_GRID = NPAD // _BR


def _dinv_from(pd_ref):
    deg = pd_ref[0, :] + pd_ref[1, :] + 1.0
    return lax.rsqrt(deg)[:, None]


def _tc_scale_matmul(x_pad, W1, pd):
    def body(x_ref, w_ref, pd_ref, o_ref):
        xw = jnp.dot(x_ref[...], w_ref[...], preferred_element_type=jnp.float32)
        o_ref[...] = xw * _dinv_from(pd_ref)

    return pl.pallas_call(
        body,
        grid=(_GRID,),
        in_specs=[
            pl.BlockSpec((_BR, 128), lambda i: (i, 0)),
            pl.BlockSpec((128, 128), lambda i: (0, 0)),
            pl.BlockSpec((2, _BR), lambda i: (0, i)),
        ],
        out_specs=pl.BlockSpec((_BR, 128), lambda i: (i, 0)),
        out_shape=jax.ShapeDtypeStruct((NPAD, 128), jnp.float32),
    )(x_pad, W1, pd)


def _tc_mid(pd, parts, xws1, b1, W2p):
    def body(pd_ref, p_ref, x_ref, b_ref, w_ref, o_ref):
        dinv = _dinv_from(pd_ref)
        h = dinv * (p_ref[0] + p_ref[1] + x_ref[...]) + b_ref[...]
        h = jnp.maximum(h, 0.0)
        o_ref[...] = jnp.dot(h, w_ref[...], preferred_element_type=jnp.float32) * dinv

    return pl.pallas_call(
        body,
        grid=(_GRID,),
        in_specs=[
            pl.BlockSpec((2, _BR), lambda i: (0, i)),
            pl.BlockSpec((2, _BR, 128), lambda i: (0, i, 0)),
            pl.BlockSpec((_BR, 128), lambda i: (i, 0)),
            pl.BlockSpec((1, 128), lambda i: (0, 0)),
            pl.BlockSpec((128, 128), lambda i: (0, 0)),
        ],
        out_specs=pl.BlockSpec((_BR, 128), lambda i: (i, 0)),
        out_shape=jax.ShapeDtypeStruct((NPAD, 128), jnp.float32),
    )(pd, parts, xws1, b1, W2p)


def _tc_final(pd, parts2, xws2, b2):
    def body(pd_ref, p_ref, x_ref, b_ref, o_ref):
        dinv = _dinv_from(pd_ref)
        zf = dinv * (p_ref[0] + p_ref[1] + x_ref[...])
        z = zf[:, :64] + b_ref[...]
        z = z - jnp.max(z, axis=1, keepdims=True)
        lse = jnp.log(jnp.sum(jnp.exp(z), axis=1, keepdims=True))
        o_ref[...] = z - lse

    return pl.pallas_call(
        body,
        grid=(_GRID,),
        in_specs=[
            pl.BlockSpec((2, _BR), lambda i: (0, i)),
            pl.BlockSpec((2, _BR, 128), lambda i: (0, i, 0)),
            pl.BlockSpec((_BR, 128), lambda i: (i, 0)),
            pl.BlockSpec((1, 64), lambda i: (0, 0)),
        ],
        out_specs=pl.BlockSpec((_BR, 64), lambda i: (i, 0)),
        out_shape=jax.ShapeDtypeStruct((NPAD, 64), jnp.float32),
    )(pd, parts2, xws2, b2)


def kernel(x, edge_index, W1, b1, W2, b2):
    src = edge_index[0].astype(jnp.int32)
    dst = edge_index[1].astype(jnp.int32)
    npad_e = EPAD - E
    pad_src = (jnp.arange(npad_e, dtype=jnp.int32) * 37) % N
    pad_dst = N + (jnp.arange(npad_e, dtype=jnp.int32) % (NPAD - N))
    src_pad = jnp.concatenate([src, pad_src])
    dst_pad = jnp.concatenate([dst, pad_dst])

    x_pad = jnp.pad(x, ((0, NPAD - N), (0, 0)))
    ones128 = jnp.ones((EB,), jnp.float32)
    zeros640 = jnp.zeros((ROWS_PER_TILE,), jnp.float32)
    z128 = jnp.zeros((EB, 128), jnp.float32)
    W2p = jnp.pad(W2, ((0, 0), (0, 64)))

    pd = _sc_deg(dst_pad, ones128, zeros640).reshape(2, NPAD)
    xws1 = _tc_scale_matmul(x_pad, W1, pd)
    p1 = _sc_agg(src_pad, dst_pad, xws1, z128, 128).reshape(2, NPAD, 128)
    xws2 = _tc_mid(pd, p1, xws1, b1.reshape(1, 128), W2p)
    p2 = _sc_agg(src_pad, dst_pad, xws2, z128, 128).reshape(2, NPAD, 128)
    out = _tc_final(pd, p2, xws2, b2.reshape(1, 64))
    return out[:N]


# speedup vs baseline: 32.1537x; 1.9898x over previous
import functools

import jax
import jax.numpy as jnp
from jax import lax
from jax.experimental import pallas as pl
from jax.experimental.pallas import tpu as pltpu
from jax.experimental.pallas import tpu_sc as plsc

N = 10000
NPAD = 10240
E = 320000
EB = 128
NTILES = 32
BLOCKS = 80
EPAD = NTILES * BLOCKS * EB
E_PER_SC = EPAD // 2
E_PER_TILE = E_PER_SC // 16
BROWS = EPAD // EB
ROWS_PER_TILE = NPAD // 16
OUT_CHUNKS = ROWS_PER_TILE // 128
NBUF = 2

_mesh = plsc.VectorSubcoreMesh(core_axis_name="c", subcore_axis_name="s")


def _sc_deg(dst2, ones128, zeros640):
    @functools.partial(
        pl.kernel,
        out_type=jax.ShapeDtypeStruct((2 * NPAD,), jnp.float32),
        mesh=_mesh,
        scratch_types=[
            pltpu.VMEM((BLOCKS, EB), jnp.int32),
            pltpu.VMEM((EB,), jnp.float32),
            pltpu.VMEM((ROWS_PER_TILE,), jnp.float32),
            pltpu.VMEM_SHARED((NPAD,), jnp.float32),
            pltpu.SemaphoreType.DMA,
        ],
    )
    def k(dst_hbm, ones_hbm, zeros_hbm, out_hbm, di_v, ones_v, buf_v, acc_sh, sem):
        c = lax.axis_index("c")
        s = lax.axis_index("s")
        row0 = pl.multiple_of(s * ROWS_PER_TILE, 128)
        tb = pl.multiple_of(c * (BROWS // 2) + s * BLOCKS, BLOCKS)
        pltpu.sync_copy(dst_hbm.at[pl.ds(tb, BLOCKS)], di_v)
        pltpu.sync_copy(ones_hbm, ones_v)
        pltpu.sync_copy(zeros_hbm, buf_v)
        pltpu.sync_copy(buf_v, acc_sh.at[pl.ds(row0, ROWS_PER_TILE)])
        plsc.subcore_barrier()

        def fire(i, carry):
            pltpu.async_copy(ones_v, acc_sh.at[di_v.at[i]], sem, add=True)
            return carry

        lax.fori_loop(0, BLOCKS, fire, 0)

        def drain(i, carry):
            pltpu.make_async_copy(ones_v, acc_sh.at[di_v.at[0]], sem).wait()
            return carry

        lax.fori_loop(0, BLOCKS, drain, 0)
        plsc.subcore_barrier()
        pltpu.sync_copy(acc_sh.at[pl.ds(row0, ROWS_PER_TILE)], buf_v)
        pltpu.sync_copy(buf_v, out_hbm.at[pl.ds(c * NPAD + row0, ROWS_PER_TILE)])

    return k(dst2, ones128, zeros640)


def _sc_agg(src2, dst2, xws, zrows):
    @functools.partial(
        pl.kernel,
        out_type=jax.ShapeDtypeStruct((2 * NPAD, 128), jnp.float32),
        mesh=_mesh,
        scratch_types=[
            pltpu.VMEM((BLOCKS, EB), jnp.int32),
            pltpu.VMEM((NBUF, EB), jnp.int32),
            pltpu.VMEM((NBUF, EB, 128), jnp.float32),
            pltpu.VMEM_SHARED((NPAD, 128), jnp.float32),
            pltpu.SemaphoreType.DMA,
            pltpu.SemaphoreType.DMA,
            pltpu.SemaphoreType.DMA,
            pltpu.SemaphoreType.DMA,
            pltpu.SemaphoreType.DMA,
            pltpu.SemaphoreType.DMA,
        ],
    )
    def k(src_hbm, dst_hbm, xws_hbm, z_hbm, out_hbm,
          si_v, di_v, rows_v, acc_sh,
          sg0, sg1, sd0, sd1, ss0, ss1):
        sg = (sg0, sg1)
        sd = (sd0, sd1)
        ss = (ss0, ss1)
        c = lax.axis_index("c")
        s = lax.axis_index("s")
        row0 = pl.multiple_of(s * ROWS_PER_TILE, 128)
        tb = pl.multiple_of(c * (BROWS // 2) + s * BLOCKS, BLOCKS)
        pltpu.sync_copy(src_hbm.at[pl.ds(tb, BLOCKS)], si_v)

        def g_start(i, j):
            pltpu.async_copy(xws_hbm.at[si_v.at[i]], rows_v.at[j], sg[j])

        def g_wait(j):
            pltpu.make_async_copy(xws_hbm.at[si_v.at[0]], rows_v.at[j], sg[j]).wait()

        def d_start(i, j):
            pltpu.async_copy(dst_hbm.at[tb + i], di_v.at[j], sd[j])

        def d_wait(j):
            pltpu.make_async_copy(dst_hbm.at[tb], di_v.at[j], sd[j]).wait()

        def s_start(j):
            pltpu.async_copy(rows_v.at[j], acc_sh.at[di_v.at[j]], ss[j], add=True)

        def s_wait(j):
            pltpu.make_async_copy(rows_v.at[j], acc_sh.at[di_v.at[0]], ss[j]).wait()

        for j in range(NBUF):
            d_start(j, j)
            g_start(j, j)

        def zinit(kk, carry):
            pltpu.sync_copy(z_hbm, acc_sh.at[pl.ds(pl.multiple_of(row0 + kk * 128, 128), 128)])
            return carry

        lax.fori_loop(0, OUT_CHUNKS, zinit, 0)
        plsc.subcore_barrier()

        def body(ii, carry):
            for j in range(NBUF):
                i = ii * NBUF + j
                jp = (j - 1) % NBUF
                @pl.when(jnp.logical_and(i >= 1, i <= BLOCKS - 2))
                def _():
                    s_wait(jp)
                    d_start(i + 1, jp)
                    g_start(i + 1, jp)

                g_wait(j)
                d_wait(j)
                s_start(j)
            return carry

        lax.fori_loop(0, BLOCKS // NBUF, body, 0)
        for j in range(NBUF):
            s_wait(j)
        plsc.subcore_barrier()

        def cout(kk, carry):
            ro = pl.multiple_of(row0 + kk * 128, 128)
            pltpu.sync_copy(acc_sh.at[pl.ds(ro, 128)], rows_v.at[0])
            pltpu.sync_copy(rows_v.at[0], out_hbm.at[pl.ds(c * NPAD + ro, 128)])
            return carry

        lax.fori_loop(0, OUT_CHUNKS, cout, 0)

    return k(src2, dst2, xws, zrows)


_BR = 1024
_GRID = NPAD // _BR


def _dinv_from(pd_ref):
    deg = pd_ref[0, :] + pd_ref[1, :] + 1.0
    return lax.rsqrt(deg)[:, None]


def _tc_scale_matmul(x_pad, W1, pd):
    def body(x_ref, w_ref, pd_ref, o_ref):
        xw = jnp.dot(x_ref[...], w_ref[...], preferred_element_type=jnp.float32)
        o_ref[...] = xw * _dinv_from(pd_ref)

    return pl.pallas_call(
        body,
        grid=(_GRID,),
        in_specs=[
            pl.BlockSpec((_BR, 128), lambda i: (i, 0)),
            pl.BlockSpec((128, 128), lambda i: (0, 0)),
            pl.BlockSpec((2, _BR), lambda i: (0, i)),
        ],
        out_specs=pl.BlockSpec((_BR, 128), lambda i: (i, 0)),
        out_shape=jax.ShapeDtypeStruct((NPAD, 128), jnp.float32),
    )(x_pad, W1, pd)


def _tc_mid(pd, parts, xws1, b1, W2p):
    def body(pd_ref, p_ref, x_ref, b_ref, w_ref, o_ref):
        dinv = _dinv_from(pd_ref)
        h = dinv * (p_ref[0] + p_ref[1] + x_ref[...]) + b_ref[...]
        h = jnp.maximum(h, 0.0)
        o_ref[...] = jnp.dot(h, w_ref[...], preferred_element_type=jnp.float32) * dinv

    return pl.pallas_call(
        body,
        grid=(_GRID,),
        in_specs=[
            pl.BlockSpec((2, _BR), lambda i: (0, i)),
            pl.BlockSpec((2, _BR, 128), lambda i: (0, i, 0)),
            pl.BlockSpec((_BR, 128), lambda i: (i, 0)),
            pl.BlockSpec((1, 128), lambda i: (0, 0)),
            pl.BlockSpec((128, 128), lambda i: (0, 0)),
        ],
        out_specs=pl.BlockSpec((_BR, 128), lambda i: (i, 0)),
        out_shape=jax.ShapeDtypeStruct((NPAD, 128), jnp.float32),
    )(pd, parts, xws1, b1, W2p)


def _tc_final(pd, parts2, xws2, b2):
    def body(pd_ref, p_ref, x_ref, b_ref, o_ref):
        dinv = _dinv_from(pd_ref)
        zf = dinv * (p_ref[0] + p_ref[1] + x_ref[...])
        z = zf[:, :64] + b_ref[...]
        z = z - jnp.max(z, axis=1, keepdims=True)
        lse = jnp.log(jnp.sum(jnp.exp(z), axis=1, keepdims=True))
        o_ref[...] = z - lse

    return pl.pallas_call(
        body,
        grid=(_GRID,),
        in_specs=[
            pl.BlockSpec((2, _BR), lambda i: (0, i)),
            pl.BlockSpec((2, _BR, 128), lambda i: (0, i, 0)),
            pl.BlockSpec((_BR, 128), lambda i: (i, 0)),
            pl.BlockSpec((1, 64), lambda i: (0, 0)),
        ],
        out_specs=pl.BlockSpec((_BR, 64), lambda i: (i, 0)),
        out_shape=jax.ShapeDtypeStruct((NPAD, 64), jnp.float32),
    )(pd, parts2, xws2, b2)


def kernel(x, edge_index, W1, b1, W2, b2):
    src = edge_index[0].astype(jnp.int32)
    dst = edge_index[1].astype(jnp.int32)
    npad_e = EPAD - E
    pad_src = (jnp.arange(npad_e, dtype=jnp.int32) * 37) % N
    pad_dst = N + (jnp.arange(npad_e, dtype=jnp.int32) % (NPAD - N))
    src2 = jnp.concatenate([src, pad_src]).reshape(BROWS, EB)
    dst2 = jnp.concatenate([dst, pad_dst]).reshape(BROWS, EB)

    x_pad = jnp.pad(x, ((0, NPAD - N), (0, 0)))
    ones128 = jnp.ones((EB,), jnp.float32)
    zeros640 = jnp.zeros((ROWS_PER_TILE,), jnp.float32)
    z128 = jnp.zeros((EB, 128), jnp.float32)
    W2p = jnp.pad(W2, ((0, 0), (0, 64)))

    pd = _sc_deg(dst2, ones128, zeros640).reshape(2, NPAD)
    xws1 = _tc_scale_matmul(x_pad, W1, pd)
    p1 = _sc_agg(src2, dst2, xws1, z128).reshape(2, NPAD, 128)
    xws2 = _tc_mid(pd, p1, xws1, b1.reshape(1, 128), W2p)
    p2 = _sc_agg(src2, dst2, xws2, z128).reshape(2, NPAD, 128)
    out = _tc_final(pd, p2, xws2, b2.reshape(1, 64))
    return out[:N]


# speedup vs baseline: 33.9696x; 1.0565x over previous
import functools

import jax
import jax.numpy as jnp
from jax import lax
from jax.experimental import pallas as pl
from jax.experimental.pallas import tpu as pltpu
from jax.experimental.pallas import tpu_sc as plsc

N = 10000
NPAD = 10240
E = 320000
EB = 128
NTILES = 32
BLOCKS = 80
EPAD = NTILES * BLOCKS * EB
E_PER_SC = EPAD // 2
E_PER_TILE = E_PER_SC // 16
BROWS = EPAD // EB
ROWS_PER_TILE = NPAD // 16
OUT_CHUNKS = ROWS_PER_TILE // 128
NBUF = 2

_mesh = plsc.VectorSubcoreMesh(core_axis_name="c", subcore_axis_name="s")


def _sc_deg(dst2, ones128, zeros640):
    @functools.partial(
        pl.kernel,
        out_type=jax.ShapeDtypeStruct((2 * NPAD,), jnp.float32),
        mesh=_mesh,
        scratch_types=[
            pltpu.VMEM((BLOCKS, EB), jnp.int32),
            pltpu.VMEM((EB,), jnp.float32),
            pltpu.VMEM((ROWS_PER_TILE,), jnp.float32),
            pltpu.VMEM_SHARED((NPAD,), jnp.float32),
            pltpu.SemaphoreType.DMA,
        ],
    )
    def k(dst_hbm, ones_hbm, zeros_hbm, out_hbm, di_v, ones_v, buf_v, acc_sh, sem):
        c = lax.axis_index("c")
        s = lax.axis_index("s")
        row0 = pl.multiple_of(s * ROWS_PER_TILE, 128)
        tb = pl.multiple_of(c * (BROWS // 2) + s * BLOCKS, BLOCKS)
        pltpu.sync_copy(dst_hbm.at[pl.ds(tb, BLOCKS)], di_v)
        pltpu.sync_copy(ones_hbm, ones_v)
        pltpu.sync_copy(zeros_hbm, buf_v)
        pltpu.sync_copy(buf_v, acc_sh.at[pl.ds(row0, ROWS_PER_TILE)])
        plsc.subcore_barrier()

        def fire(i, carry):
            pltpu.async_copy(ones_v, acc_sh.at[di_v.at[i]], sem, add=True)
            return carry

        lax.fori_loop(0, BLOCKS, fire, 0)

        def drain(i, carry):
            pltpu.make_async_copy(ones_v, acc_sh.at[di_v.at[0]], sem).wait()
            return carry

        lax.fori_loop(0, BLOCKS, drain, 0)
        plsc.subcore_barrier()
        pltpu.sync_copy(acc_sh.at[pl.ds(row0, ROWS_PER_TILE)], buf_v)
        pltpu.sync_copy(buf_v, out_hbm.at[pl.ds(c * NPAD + row0, ROWS_PER_TILE)])

    return k(dst2, ones128, zeros640)


def _sc_agg(src1, dst1, xws, zrows):
    EBA = 80
    BLK = E_PER_TILE // EBA
    NB = 4
    LA = 2

    @functools.partial(
        pl.kernel,
        out_type=jax.ShapeDtypeStruct((2 * NPAD, 128), jnp.float32),
        mesh=_mesh,
        scratch_types=[
            pltpu.VMEM((NB, EBA), jnp.int32),
            pltpu.VMEM((NB, EBA), jnp.int32),
            pltpu.VMEM((NB, EBA, 128), jnp.float32),
            pltpu.VMEM_SHARED((NPAD, 128), jnp.float32),
            pltpu.SemaphoreType.DMA, pltpu.SemaphoreType.DMA,
            pltpu.SemaphoreType.DMA, pltpu.SemaphoreType.DMA,
            pltpu.SemaphoreType.DMA, pltpu.SemaphoreType.DMA,
            pltpu.SemaphoreType.DMA, pltpu.SemaphoreType.DMA,
            pltpu.SemaphoreType.DMA, pltpu.SemaphoreType.DMA,
            pltpu.SemaphoreType.DMA, pltpu.SemaphoreType.DMA,
            pltpu.SemaphoreType.DMA, pltpu.SemaphoreType.DMA,
            pltpu.SemaphoreType.DMA, pltpu.SemaphoreType.DMA,
        ],
    )
    def k(src_hbm, dst_hbm, xws_hbm, z_hbm, out_hbm,
          si_v, di_v, rows_v, acc_sh,
          sg0, sg1, sg2, sg3, si0, si1, si2, si3,
          sd0, sd1, sd2, sd3, ss0, ss1, ss2, ss3):
        sg = (sg0, sg1, sg2, sg3)
        sis = (si0, si1, si2, si3)
        sd = (sd0, sd1, sd2, sd3)
        ss = (ss0, ss1, ss2, ss3)
        c = lax.axis_index("c")
        s = lax.axis_index("s")
        row0 = pl.multiple_of(s * ROWS_PER_TILE, 128)
        eb0 = pl.multiple_of((c * 16 + s) * E_PER_TILE, EBA)

        def i_start(i, j):
            pltpu.async_copy(src_hbm.at[pl.ds(pl.multiple_of(eb0 + i * EBA, EBA), EBA)],
                             si_v.at[j], sis[j])

        def i_wait(j):
            pltpu.make_async_copy(src_hbm.at[pl.ds(eb0, EBA)], si_v.at[j], sis[j]).wait()

        def g_start(j):
            pltpu.async_copy(xws_hbm.at[si_v.at[j]], rows_v.at[j], sg[j])

        def g_wait(j):
            pltpu.make_async_copy(xws_hbm.at[si_v.at[0]], rows_v.at[j], sg[j]).wait()

        def d_start(i, j):
            pltpu.async_copy(dst_hbm.at[pl.ds(pl.multiple_of(eb0 + i * EBA, EBA), EBA)],
                             di_v.at[j], sd[j])

        def d_wait(j):
            pltpu.make_async_copy(dst_hbm.at[pl.ds(eb0, EBA)], di_v.at[j], sd[j]).wait()

        def s_start(j):
            pltpu.async_copy(rows_v.at[j], acc_sh.at[di_v.at[j]], ss[j], add=True)

        def s_wait(j):
            pltpu.make_async_copy(rows_v.at[j], acc_sh.at[di_v.at[0]], ss[j]).wait()

        for j in range(LA + 1):
            i_start(j, j)
        for j in range(LA):
            d_start(j, j)
        for j in range(LA):
            i_wait(j)
            g_start(j)

        def zinit(kk, carry):
            pltpu.sync_copy(z_hbm, acc_sh.at[pl.ds(pl.multiple_of(row0 + kk * 128, 128), 128)])
            return carry

        lax.fori_loop(0, OUT_CHUNKS, zinit, 0)
        plsc.subcore_barrier()

        def body(ii, carry):
            for j in range(NB):
                i = ii * NB + j
                ja = (j + LA) % NB
                jb = (j + LA + 1) % NB

                @pl.when(i + LA <= BLK - 1)
                def _():
                    @pl.when(i - LA >= 0)
                    def _():
                        s_wait(ja)
                    d_start(i + LA, ja)

                @pl.when(i + LA + 1 <= BLK - 1)
                def _():
                    i_start(i + LA + 1, jb)

                @pl.when(i + LA <= BLK - 1)
                def _():
                    i_wait(ja)
                    g_start(ja)

                g_wait(j)
                d_wait(j)
                s_start(j)
            return carry

        lax.fori_loop(0, BLK // NB, body, 0)
        for j in range(NB):
            s_wait(j)
        plsc.subcore_barrier()

        def cout(kk, carry):
            ro = pl.multiple_of(row0 + kk * EBA, 16)
            pltpu.sync_copy(acc_sh.at[pl.ds(ro, EBA)], rows_v.at[0])
            pltpu.sync_copy(rows_v.at[0], out_hbm.at[pl.ds(c * NPAD + ro, EBA)])
            return carry

        lax.fori_loop(0, ROWS_PER_TILE // EBA, cout, 0)

    return k(src1, dst1, xws, zrows)


_BR = 1024
_GRID = NPAD // _BR


def _dinv_from(pd_ref):
    deg = pd_ref[0, :] + pd_ref[1, :] + 1.0
    return lax.rsqrt(deg)[:, None]


def _tc_scale_matmul(x_pad, W1, pd):
    def body(x_ref, w_ref, pd_ref, o_ref):
        xw = jnp.dot(x_ref[...], w_ref[...], preferred_element_type=jnp.float32)
        o_ref[...] = xw * _dinv_from(pd_ref)

    return pl.pallas_call(
        body,
        grid=(_GRID,),
        in_specs=[
            pl.BlockSpec((_BR, 128), lambda i: (i, 0)),
            pl.BlockSpec((128, 128), lambda i: (0, 0)),
            pl.BlockSpec((2, _BR), lambda i: (0, i)),
        ],
        out_specs=pl.BlockSpec((_BR, 128), lambda i: (i, 0)),
        out_shape=jax.ShapeDtypeStruct((NPAD, 128), jnp.float32),
    )(x_pad, W1, pd)


def _tc_mid(pd, parts, xws1, b1, W2p):
    def body(pd_ref, p_ref, x_ref, b_ref, w_ref, o_ref):
        dinv = _dinv_from(pd_ref)
        h = dinv * (p_ref[0] + p_ref[1] + x_ref[...]) + b_ref[...]
        h = jnp.maximum(h, 0.0)
        o_ref[...] = jnp.dot(h, w_ref[...], preferred_element_type=jnp.float32) * dinv

    return pl.pallas_call(
        body,
        grid=(_GRID,),
        in_specs=[
            pl.BlockSpec((2, _BR), lambda i: (0, i)),
            pl.BlockSpec((2, _BR, 128), lambda i: (0, i, 0)),
            pl.BlockSpec((_BR, 128), lambda i: (i, 0)),
            pl.BlockSpec((1, 128), lambda i: (0, 0)),
            pl.BlockSpec((128, 128), lambda i: (0, 0)),
        ],
        out_specs=pl.BlockSpec((_BR, 128), lambda i: (i, 0)),
        out_shape=jax.ShapeDtypeStruct((NPAD, 128), jnp.float32),
    )(pd, parts, xws1, b1, W2p)


def _tc_final(pd, parts2, xws2, b2):
    def body(pd_ref, p_ref, x_ref, b_ref, o_ref):
        dinv = _dinv_from(pd_ref)
        zf = dinv * (p_ref[0] + p_ref[1] + x_ref[...])
        z = zf[:, :64] + b_ref[...]
        z = z - jnp.max(z, axis=1, keepdims=True)
        lse = jnp.log(jnp.sum(jnp.exp(z), axis=1, keepdims=True))
        o_ref[...] = z - lse

    return pl.pallas_call(
        body,
        grid=(_GRID,),
        in_specs=[
            pl.BlockSpec((2, _BR), lambda i: (0, i)),
            pl.BlockSpec((2, _BR, 128), lambda i: (0, i, 0)),
            pl.BlockSpec((_BR, 128), lambda i: (i, 0)),
            pl.BlockSpec((1, 64), lambda i: (0, 0)),
        ],
        out_specs=pl.BlockSpec((_BR, 64), lambda i: (i, 0)),
        out_shape=jax.ShapeDtypeStruct((NPAD, 64), jnp.float32),
    )(pd, parts2, xws2, b2)


def kernel(x, edge_index, W1, b1, W2, b2):
    src = edge_index[0].astype(jnp.int32)
    dst = edge_index[1].astype(jnp.int32)
    npad_e = EPAD - E
    pad_src = (jnp.arange(npad_e, dtype=jnp.int32) * 37) % N
    pad_dst = N + (jnp.arange(npad_e, dtype=jnp.int32) % (NPAD - N))
    src1 = jnp.concatenate([src, pad_src])
    dst1 = jnp.concatenate([dst, pad_dst])
    dst2 = dst1.reshape(BROWS, EB)

    x_pad = jnp.pad(x, ((0, NPAD - N), (0, 0)))
    ones128 = jnp.ones((EB,), jnp.float32)
    zeros640 = jnp.zeros((ROWS_PER_TILE,), jnp.float32)
    z128 = jnp.zeros((EB, 128), jnp.float32)
    W2p = jnp.pad(W2, ((0, 0), (0, 64)))

    pd = _sc_deg(dst2, ones128, zeros640).reshape(2, NPAD)
    xws1 = _tc_scale_matmul(x_pad, W1, pd)
    p1 = _sc_agg(src1, dst1, xws1, z128).reshape(2, NPAD, 128)
    xws2 = _tc_mid(pd, p1, xws1, b1.reshape(1, 128), W2p)
    p2 = _sc_agg(src1, dst1, xws2, z128).reshape(2, NPAD, 128)
    out = _tc_final(pd, p2, xws2, b2.reshape(1, 64))
    return out[:N]


# speedup vs baseline: 34.0703x; 1.0030x over previous
import functools

import jax
import jax.numpy as jnp
from jax import lax
from jax.experimental import pallas as pl
from jax.experimental.pallas import tpu as pltpu
from jax.experimental.pallas import tpu_sc as plsc

N = 10000
NPAD = 10240
E = 320000
EB = 128
NTILES = 32
BLOCKS = 80
EPAD = NTILES * BLOCKS * EB
E_PER_SC = EPAD // 2
E_PER_TILE = E_PER_SC // 16
BROWS = EPAD // EB
ROWS_PER_TILE = NPAD // 16
OUT_CHUNKS = ROWS_PER_TILE // 128
NBUF = 2

_mesh = plsc.VectorSubcoreMesh(core_axis_name="c", subcore_axis_name="s")


def _sc_deg(dst2, ones128, zeros640):
    @functools.partial(
        pl.kernel,
        out_type=jax.ShapeDtypeStruct((2 * NPAD,), jnp.float32),
        mesh=_mesh,
        scratch_types=[
            pltpu.VMEM((BLOCKS, EB), jnp.int32),
            pltpu.VMEM((EB,), jnp.float32),
            pltpu.VMEM((ROWS_PER_TILE,), jnp.float32),
            pltpu.VMEM_SHARED((NPAD,), jnp.float32),
            pltpu.SemaphoreType.DMA,
        ],
    )
    def k(dst_hbm, ones_hbm, zeros_hbm, out_hbm, di_v, ones_v, buf_v, acc_sh, sem):
        c = lax.axis_index("c")
        s = lax.axis_index("s")
        row0 = pl.multiple_of(s * ROWS_PER_TILE, 128)
        tb = pl.multiple_of(c * (BROWS // 2) + s * BLOCKS, BLOCKS)
        pltpu.sync_copy(dst_hbm.at[pl.ds(tb, BLOCKS)], di_v)
        pltpu.sync_copy(ones_hbm, ones_v)
        pltpu.sync_copy(zeros_hbm, buf_v)
        pltpu.sync_copy(buf_v, acc_sh.at[pl.ds(row0, ROWS_PER_TILE)])
        plsc.subcore_barrier()

        def fire(i, carry):
            pltpu.async_copy(ones_v, acc_sh.at[di_v.at[i]], sem, add=True)
            return carry

        lax.fori_loop(0, BLOCKS, fire, 0)

        def drain(i, carry):
            pltpu.make_async_copy(ones_v, acc_sh.at[di_v.at[0]], sem).wait()
            return carry

        lax.fori_loop(0, BLOCKS, drain, 0)
        plsc.subcore_barrier()
        pltpu.sync_copy(acc_sh.at[pl.ds(row0, ROWS_PER_TILE)], buf_v)
        pltpu.sync_copy(buf_v, out_hbm.at[pl.ds(c * NPAD + row0, ROWS_PER_TILE)])

    return k(dst2, ones128, zeros640)


def _sc_agg(src1, dst1, xws, zrows):
    EBA = 80
    BLK = E_PER_TILE // EBA
    NB = 4
    LA = 2

    @functools.partial(
        pl.kernel,
        out_type=jax.ShapeDtypeStruct((2 * NPAD, 128), jnp.float32),
        mesh=_mesh,
        scratch_types=[
            pltpu.VMEM((NB, EBA), jnp.int32),
            pltpu.VMEM((NB, EBA), jnp.int32),
            pltpu.VMEM((NB, EBA, 128), jnp.float32),
            pltpu.VMEM_SHARED((NPAD, 128), jnp.float32),
            pltpu.SemaphoreType.DMA, pltpu.SemaphoreType.DMA,
            pltpu.SemaphoreType.DMA, pltpu.SemaphoreType.DMA,
            pltpu.SemaphoreType.DMA, pltpu.SemaphoreType.DMA,
            pltpu.SemaphoreType.DMA, pltpu.SemaphoreType.DMA,
            pltpu.SemaphoreType.DMA, pltpu.SemaphoreType.DMA,
            pltpu.SemaphoreType.DMA, pltpu.SemaphoreType.DMA,
            pltpu.SemaphoreType.DMA, pltpu.SemaphoreType.DMA,
            pltpu.SemaphoreType.DMA, pltpu.SemaphoreType.DMA,
        ],
    )
    def k(src_hbm, dst_hbm, xws_hbm, z_hbm, out_hbm,
          si_v, di_v, rows_v, acc_sh,
          sg0, sg1, sg2, sg3, si0, si1, si2, si3,
          sd0, sd1, sd2, sd3, ss0, ss1, ss2, ss3):
        sg = (sg0, sg1, sg2, sg3)
        sis = (si0, si1, si2, si3)
        sd = (sd0, sd1, sd2, sd3)
        ss = (ss0, ss1, ss2, ss3)
        c = lax.axis_index("c")
        s = lax.axis_index("s")
        row0 = pl.multiple_of(s * ROWS_PER_TILE, 128)
        eb0 = pl.multiple_of((c * 16 + s) * E_PER_TILE, EBA)

        def i_start(i, j):
            pltpu.async_copy(src_hbm.at[pl.ds(pl.multiple_of(eb0 + i * EBA, EBA), EBA)],
                             si_v.at[j], sis[j])

        def i_wait(j):
            pltpu.make_async_copy(src_hbm.at[pl.ds(eb0, EBA)], si_v.at[j], sis[j]).wait()

        def g_start(j):
            pltpu.async_copy(xws_hbm.at[si_v.at[j]], rows_v.at[j], sg[j])

        def g_wait(j):
            pltpu.make_async_copy(xws_hbm.at[si_v.at[0]], rows_v.at[j], sg[j]).wait()

        def d_start(i, j):
            pltpu.async_copy(dst_hbm.at[pl.ds(pl.multiple_of(eb0 + i * EBA, EBA), EBA)],
                             di_v.at[j], sd[j])

        def d_wait(j):
            pltpu.make_async_copy(dst_hbm.at[pl.ds(eb0, EBA)], di_v.at[j], sd[j]).wait()

        def s_start(j):
            pltpu.async_copy(rows_v.at[j], acc_sh.at[di_v.at[j]], ss[j], add=True)

        def s_wait(j):
            pltpu.make_async_copy(rows_v.at[j], acc_sh.at[di_v.at[0]], ss[j]).wait()

        for j in range(LA + 1):
            i_start(j, j)
        for j in range(LA):
            d_start(j, j)
        for j in range(LA):
            i_wait(j)
            g_start(j)

        def zinit(kk, carry):
            pltpu.sync_copy(z_hbm, acc_sh.at[pl.ds(pl.multiple_of(row0 + kk * 128, 128), 128)])
            return carry

        lax.fori_loop(0, OUT_CHUNKS, zinit, 0)
        plsc.subcore_barrier()

        def body(ii, carry):
            for j in range(NB):
                i = ii * NB + j
                ja = (j + LA) % NB
                jb = (j + LA + 1) % NB

                @pl.when(i + LA <= BLK - 1)
                def _():
                    @pl.when(i - LA >= 0)
                    def _():
                        s_wait(ja)
                    d_start(i + LA, ja)

                @pl.when(i + LA + 1 <= BLK - 1)
                def _():
                    i_start(i + LA + 1, jb)

                @pl.when(i + LA <= BLK - 1)
                def _():
                    i_wait(ja)
                    g_start(ja)

                g_wait(j)
                d_wait(j)
                s_start(j)
            return carry

        lax.fori_loop(0, BLK // NB, body, 0)
        for j in range(NB):
            s_wait(j)
        plsc.subcore_barrier()

        def cout(kk, carry):
            ro = pl.multiple_of(row0 + kk * EBA, 16)
            pltpu.sync_copy(acc_sh.at[pl.ds(ro, EBA)], rows_v.at[0])
            pltpu.sync_copy(rows_v.at[0], out_hbm.at[pl.ds(c * NPAD + ro, EBA)])
            return carry

        lax.fori_loop(0, ROWS_PER_TILE // EBA, cout, 0)

    return k(src1, dst1, xws, zrows)


_BR = 1024
_GRID = NPAD // _BR


def _dinv_from(pd_ref):
    deg = pd_ref[0, :] + pd_ref[1, :] + 1.0
    return lax.rsqrt(deg)[:, None]


def _tc_scale_matmul(x, W1, pd):
    def body(x_ref, w_ref, pd_ref, o_ref):
        xw = jnp.dot(x_ref[...], w_ref[...], preferred_element_type=jnp.float32)
        o_ref[...] = xw * _dinv_from(pd_ref)

    return pl.pallas_call(
        body,
        grid=(_GRID,),
        in_specs=[
            pl.BlockSpec((_BR, 128), lambda i: (i, 0)),
            pl.BlockSpec((128, 128), lambda i: (0, 0)),
            pl.BlockSpec((2, _BR), lambda i: (0, i)),
        ],
        out_specs=pl.BlockSpec((_BR, 128), lambda i: (i, 0)),
        out_shape=jax.ShapeDtypeStruct((NPAD, 128), jnp.float32),
    )(x, W1, pd)


def _tc_mid(pd, parts, xws1, b1, W2p):
    def body(pd_ref, p_ref, x_ref, b_ref, w_ref, o_ref):
        dinv = _dinv_from(pd_ref)
        h = dinv * (p_ref[0] + p_ref[1] + x_ref[...]) + b_ref[...]
        h = jnp.maximum(h, 0.0)
        o_ref[...] = jnp.dot(h, w_ref[...], preferred_element_type=jnp.float32) * dinv

    return pl.pallas_call(
        body,
        grid=(_GRID,),
        in_specs=[
            pl.BlockSpec((2, _BR), lambda i: (0, i)),
            pl.BlockSpec((2, _BR, 128), lambda i: (0, i, 0)),
            pl.BlockSpec((_BR, 128), lambda i: (i, 0)),
            pl.BlockSpec((1, 128), lambda i: (0, 0)),
            pl.BlockSpec((128, 128), lambda i: (0, 0)),
        ],
        out_specs=pl.BlockSpec((_BR, 128), lambda i: (i, 0)),
        out_shape=jax.ShapeDtypeStruct((NPAD, 128), jnp.float32),
    )(pd, parts, xws1, b1, W2p)


def _tc_final(pd, parts2, xws2, b2):
    def body(pd_ref, p_ref, x_ref, b_ref, o_ref):
        dinv = _dinv_from(pd_ref)
        zf = dinv * (p_ref[0] + p_ref[1] + x_ref[...])
        z = zf[:, :64] + b_ref[...]
        z = z - jnp.max(z, axis=1, keepdims=True)
        lse = jnp.log(jnp.sum(jnp.exp(z), axis=1, keepdims=True))
        o_ref[...] = z - lse

    return pl.pallas_call(
        body,
        grid=(_GRID,),
        in_specs=[
            pl.BlockSpec((2, _BR), lambda i: (0, i)),
            pl.BlockSpec((2, _BR, 128), lambda i: (0, i, 0)),
            pl.BlockSpec((_BR, 128), lambda i: (i, 0)),
            pl.BlockSpec((1, 64), lambda i: (0, 0)),
        ],
        out_specs=pl.BlockSpec((_BR, 64), lambda i: (i, 0)),
        out_shape=jax.ShapeDtypeStruct((N, 64), jnp.float32),
    )(pd, parts2, xws2, b2)


def kernel(x, edge_index, W1, b1, W2, b2):
    src = edge_index[0].astype(jnp.int32)
    dst = edge_index[1].astype(jnp.int32)
    npad_e = EPAD - E
    pad_src = (jnp.arange(npad_e, dtype=jnp.int32) * 37) % N
    pad_dst = N + (jnp.arange(npad_e, dtype=jnp.int32) % (NPAD - N))
    src1 = jnp.concatenate([src, pad_src])
    dst1 = jnp.concatenate([dst, pad_dst])
    dst2 = dst1.reshape(BROWS, EB)

    ones128 = jnp.ones((EB,), jnp.float32)
    zeros640 = jnp.zeros((ROWS_PER_TILE,), jnp.float32)
    z128 = jnp.zeros((EB, 128), jnp.float32)
    W2p = jnp.pad(W2, ((0, 0), (0, 64)))

    pd = _sc_deg(dst2, ones128, zeros640).reshape(2, NPAD)
    xws1 = _tc_scale_matmul(x, W1, pd)
    p1 = _sc_agg(src1, dst1, xws1, z128).reshape(2, NPAD, 128)
    xws2 = _tc_mid(pd, p1, xws1, b1.reshape(1, 128), W2p)
    p2 = _sc_agg(src1, dst1, xws2, z128).reshape(2, NPAD, 128)
    return _tc_final(pd, p2, xws2, b2.reshape(1, 64))


# speedup vs baseline: 34.8401x; 1.0226x over previous
import functools

import jax
import jax.numpy as jnp
from jax import lax
from jax.experimental import pallas as pl
from jax.experimental.pallas import tpu as pltpu
from jax.experimental.pallas import tpu_sc as plsc

N = 10000
NPAD = 10240
E = 320000
EB = 128
NTILES = 32
BLOCKS = 80
EPAD = NTILES * BLOCKS * EB
E_PER_SC = EPAD // 2
E_PER_TILE = E_PER_SC // 16
BROWS = EPAD // EB
ROWS_PER_TILE = NPAD // 16
OUT_CHUNKS = ROWS_PER_TILE // 128
NBUF = 2

_mesh = plsc.VectorSubcoreMesh(core_axis_name="c", subcore_axis_name="s")


def _sc_deg(dst2, ones128, zeros640):
    @functools.partial(
        pl.kernel,
        out_type=jax.ShapeDtypeStruct((2 * NPAD,), jnp.float32),
        mesh=_mesh,
        scratch_types=[
            pltpu.VMEM((BLOCKS, EB), jnp.int32),
            pltpu.VMEM((EB,), jnp.float32),
            pltpu.VMEM((ROWS_PER_TILE,), jnp.float32),
            pltpu.VMEM_SHARED((NPAD,), jnp.float32),
            pltpu.SemaphoreType.DMA,
        ],
    )
    def k(dst_hbm, ones_hbm, zeros_hbm, out_hbm, di_v, ones_v, buf_v, acc_sh, sem):
        c = lax.axis_index("c")
        s = lax.axis_index("s")
        row0 = pl.multiple_of(s * ROWS_PER_TILE, 128)
        tb = pl.multiple_of(c * (BROWS // 2) + s * BLOCKS, BLOCKS)
        pltpu.sync_copy(dst_hbm.at[pl.ds(tb, BLOCKS)], di_v)
        pltpu.sync_copy(ones_hbm, ones_v)
        pltpu.sync_copy(zeros_hbm, buf_v)
        pltpu.sync_copy(buf_v, acc_sh.at[pl.ds(row0, ROWS_PER_TILE)])
        plsc.subcore_barrier()

        def fire(i, carry):
            pltpu.async_copy(ones_v, acc_sh.at[di_v.at[i]], sem, add=True)
            return carry

        lax.fori_loop(0, BLOCKS, fire, 0)

        def drain(i, carry):
            pltpu.make_async_copy(ones_v, acc_sh.at[di_v.at[0]], sem).wait()
            return carry

        lax.fori_loop(0, BLOCKS, drain, 0)
        plsc.subcore_barrier()
        pltpu.sync_copy(acc_sh.at[pl.ds(row0, ROWS_PER_TILE)], buf_v)
        pltpu.sync_copy(buf_v, out_hbm.at[pl.ds(c * NPAD + row0, ROWS_PER_TILE)])

    return k(dst2, ones128, zeros640)


def _sc_agg(src1, dst1, xws, zrows):
    EBA = 80
    BLK = E_PER_TILE // EBA
    NB = 4
    LA = 2

    @functools.partial(
        pl.kernel,
        out_type=jax.ShapeDtypeStruct((2 * NPAD, 128), jnp.float32),
        mesh=_mesh,
        scratch_types=[
            pltpu.VMEM((NB, EBA), jnp.int32),
            pltpu.VMEM((NB, EBA), jnp.int32),
            pltpu.VMEM((NB, EBA, 128), jnp.float32),
            pltpu.VMEM_SHARED((NPAD, 128), jnp.float32),
            pltpu.SemaphoreType.DMA, pltpu.SemaphoreType.DMA,
            pltpu.SemaphoreType.DMA, pltpu.SemaphoreType.DMA,
            pltpu.SemaphoreType.DMA, pltpu.SemaphoreType.DMA,
            pltpu.SemaphoreType.DMA, pltpu.SemaphoreType.DMA,
            pltpu.SemaphoreType.DMA, pltpu.SemaphoreType.DMA,
            pltpu.SemaphoreType.DMA, pltpu.SemaphoreType.DMA,
            pltpu.SemaphoreType.DMA, pltpu.SemaphoreType.DMA,
            pltpu.SemaphoreType.DMA, pltpu.SemaphoreType.DMA,
        ],
    )
    def k(src_hbm, dst_hbm, xws_hbm, z_hbm, out_hbm,
          si_v, di_v, rows_v, acc_sh,
          sg0, sg1, sg2, sg3, si0, si1, si2, si3,
          sd0, sd1, sd2, sd3, ss0, ss1, ss2, ss3):
        sg = (sg0, sg1, sg2, sg3)
        sis = (si0, si1, si2, si3)
        sd = (sd0, sd1, sd2, sd3)
        ss = (ss0, ss1, ss2, ss3)
        c = lax.axis_index("c")
        s = lax.axis_index("s")
        row0 = pl.multiple_of(s * ROWS_PER_TILE, 128)
        eb0 = pl.multiple_of((c * 16 + s) * E_PER_TILE, EBA)

        def i_start(i, j):
            pltpu.async_copy(src_hbm.at[pl.ds(pl.multiple_of(eb0 + i * EBA, EBA), EBA)],
                             si_v.at[j], sis[j])

        def i_wait(j):
            pltpu.make_async_copy(src_hbm.at[pl.ds(eb0, EBA)], si_v.at[j], sis[j]).wait()

        def g_start(j):
            pltpu.async_copy(xws_hbm.at[si_v.at[j]], rows_v.at[j], sg[j])

        def g_wait(j):
            pltpu.make_async_copy(xws_hbm.at[si_v.at[0]], rows_v.at[j], sg[j]).wait()

        def d_start(i, j):
            pltpu.async_copy(dst_hbm.at[pl.ds(pl.multiple_of(eb0 + i * EBA, EBA), EBA)],
                             di_v.at[j], sd[j])

        def d_wait(j):
            pltpu.make_async_copy(dst_hbm.at[pl.ds(eb0, EBA)], di_v.at[j], sd[j]).wait()

        def s_start(j):
            pltpu.async_copy(rows_v.at[j], acc_sh.at[di_v.at[j]], ss[j], add=True)

        def s_wait(j):
            pltpu.make_async_copy(rows_v.at[j], acc_sh.at[di_v.at[0]], ss[j]).wait()

        for j in range(LA + 1):
            i_start(j, j)
        for j in range(LA):
            d_start(j, j)
        for j in range(LA):
            i_wait(j)
            g_start(j)

        def zinit(kk, carry):
            pltpu.sync_copy(z_hbm, acc_sh.at[pl.ds(pl.multiple_of(row0 + kk * 128, 128), 128)])
            return carry

        lax.fori_loop(0, OUT_CHUNKS, zinit, 0)
        plsc.subcore_barrier()

        def body(ii, carry):
            for j in range(NB):
                i = ii * NB + j
                ja = (j + LA) % NB
                jb = (j + LA + 1) % NB

                @pl.when(i + LA <= BLK - 1)
                def _():
                    @pl.when(i - LA >= 0)
                    def _():
                        s_wait(ja)
                    d_start(i + LA, ja)

                @pl.when(i + LA + 1 <= BLK - 1)
                def _():
                    i_start(i + LA + 1, jb)

                @pl.when(i + LA <= BLK - 1)
                def _():
                    i_wait(ja)
                    g_start(ja)

                g_wait(j)
                d_wait(j)
                s_start(j)
            return carry

        lax.fori_loop(0, BLK // NB, body, 0)
        for j in range(NB):
            s_wait(j)
        plsc.subcore_barrier()

        def cout(kk, carry):
            ro = pl.multiple_of(row0 + kk * EBA, 16)
            pltpu.sync_copy(acc_sh.at[pl.ds(ro, EBA)], rows_v.at[0])
            pltpu.sync_copy(rows_v.at[0], out_hbm.at[pl.ds(c * NPAD + ro, EBA)])
            return carry

        lax.fori_loop(0, ROWS_PER_TILE // EBA, cout, 0)

    return k(src1, dst1, xws, zrows)


_BR = 2048
_GRID = NPAD // _BR


def _dinv_from(pd_ref):
    deg = pd_ref[0, :] + pd_ref[1, :] + 1.0
    return lax.rsqrt(deg)[:, None]


def _tc_scale_matmul(x, W1, pd):
    def body(x_ref, w_ref, pd_ref, o_ref):
        xw = jnp.dot(x_ref[...], w_ref[...], preferred_element_type=jnp.float32)
        o_ref[...] = xw * _dinv_from(pd_ref)

    return pl.pallas_call(
        body,
        grid=(_GRID,),
        in_specs=[
            pl.BlockSpec((_BR, 128), lambda i: (i, 0)),
            pl.BlockSpec((128, 128), lambda i: (0, 0)),
            pl.BlockSpec((2, _BR), lambda i: (0, i)),
        ],
        out_specs=pl.BlockSpec((_BR, 128), lambda i: (i, 0)),
        out_shape=jax.ShapeDtypeStruct((NPAD, 128), jnp.float32),
    )(x, W1, pd)


def _tc_mid(pd, parts, xws1, b1, W2p):
    def body(pd_ref, p_ref, x_ref, b_ref, w_ref, o_ref):
        dinv = _dinv_from(pd_ref)
        h = dinv * (p_ref[0] + p_ref[1] + x_ref[...]) + b_ref[...]
        h = jnp.maximum(h, 0.0)
        o_ref[...] = jnp.dot(h, w_ref[...], preferred_element_type=jnp.float32) * dinv

    return pl.pallas_call(
        body,
        grid=(_GRID,),
        in_specs=[
            pl.BlockSpec((2, _BR), lambda i: (0, i)),
            pl.BlockSpec((2, _BR, 128), lambda i: (0, i, 0)),
            pl.BlockSpec((_BR, 128), lambda i: (i, 0)),
            pl.BlockSpec((1, 128), lambda i: (0, 0)),
            pl.BlockSpec((128, 128), lambda i: (0, 0)),
        ],
        out_specs=pl.BlockSpec((_BR, 128), lambda i: (i, 0)),
        out_shape=jax.ShapeDtypeStruct((NPAD, 128), jnp.float32),
    )(pd, parts, xws1, b1, W2p)


def _tc_final(pd, parts2, xws2, b2):
    def body(pd_ref, p_ref, x_ref, b_ref, o_ref):
        dinv = _dinv_from(pd_ref)
        zf = dinv * (p_ref[0] + p_ref[1] + x_ref[...])
        z = zf[:, :64] + b_ref[...]
        z = z - jnp.max(z, axis=1, keepdims=True)
        lse = jnp.log(jnp.sum(jnp.exp(z), axis=1, keepdims=True))
        o_ref[...] = z - lse

    return pl.pallas_call(
        body,
        grid=(_GRID,),
        in_specs=[
            pl.BlockSpec((2, _BR), lambda i: (0, i)),
            pl.BlockSpec((2, _BR, 128), lambda i: (0, i, 0)),
            pl.BlockSpec((_BR, 128), lambda i: (i, 0)),
            pl.BlockSpec((1, 64), lambda i: (0, 0)),
        ],
        out_specs=pl.BlockSpec((_BR, 64), lambda i: (i, 0)),
        out_shape=jax.ShapeDtypeStruct((N, 64), jnp.float32),
    )(pd, parts2, xws2, b2)


def kernel(x, edge_index, W1, b1, W2, b2):
    src = edge_index[0].astype(jnp.int32)
    dst = edge_index[1].astype(jnp.int32)
    npad_e = EPAD - E
    pad_src = (jnp.arange(npad_e, dtype=jnp.int32) * 37) % N
    pad_dst = N + (jnp.arange(npad_e, dtype=jnp.int32) % (NPAD - N))
    src1 = jnp.concatenate([src, pad_src])
    dst1 = jnp.concatenate([dst, pad_dst])
    dst2 = dst1.reshape(BROWS, EB)

    ones128 = jnp.ones((EB,), jnp.float32)
    zeros640 = jnp.zeros((ROWS_PER_TILE,), jnp.float32)
    z128 = jnp.zeros((EB, 128), jnp.float32)
    W2p = jnp.pad(W2, ((0, 0), (0, 64)))

    pd = _sc_deg(dst2, ones128, zeros640).reshape(2, NPAD)
    xws1 = _tc_scale_matmul(x, W1, pd)
    p1 = _sc_agg(src1, dst1, xws1, z128).reshape(2, NPAD, 128)
    xws2 = _tc_mid(pd, p1, xws1, b1.reshape(1, 128), W2p)
    p2 = _sc_agg(src1, dst1, xws2, z128).reshape(2, NPAD, 128)
    return _tc_final(pd, p2, xws2, b2.reshape(1, 64))
